# baseline (speedup 1.0000x reference)
import functools

import jax
import jax.numpy as jnp
from jax import lax
from jax.experimental import pallas as pl
from jax.experimental.pallas import tpu as pltpu
from jax.experimental.pallas import tpu_sc as plsc

NC = 2
NS = 16
NW = NC * NS
K = 100
SZ = 80
NPH = 20


def _rup(a, m):
    return (a + m - 1) // m * m


def _make_seg_kernel(N, C, NP, RT, EPT, NCH):
    mesh = plsc.VectorSubcoreMesh(core_axis_name="c", subcore_axis_name="s")
    PHN = NCH // NPH

    @functools.partial(
        pl.kernel,
        mesh=mesh,
        out_type=jax.ShapeDtypeStruct((NC, NP, C), jnp.float32),
        scratch_types=[
            pltpu.VMEM((NPH, K), jnp.int32),
            pltpu.VMEM((NPH, K), jnp.int32),
            pltpu.VMEM((K, C), jnp.float32),
            pltpu.VMEM((K, C), jnp.float32),
            pltpu.VMEM((K, C), jnp.float32),
            pltpu.VMEM_SHARED((NP, C), jnp.float32),
            pltpu.SemaphoreType.DMA,
            pltpu.SemaphoreType.DMA,
            pltpu.SemaphoreType.DMA,
            pltpu.SemaphoreType.DMA,
            pltpu.SemaphoreType.DMA,
            pltpu.SemaphoreType.DMA,
        ],
    )
    def seg(h_hbm, src_hbm, dst_hbm, z_hbm, out_hbm, srcs_v, dsts_v,
            r0, r1, r2, acc, g0, g1, g2, s0, s1, s2):
        c = lax.axis_index("c")
        s = lax.axis_index("s")
        wid = s * NC + c
        R = (r0, r1, r2)
        GS = (g0, g1, g2)
        SS = (s0, s1, s2)
        pltpu.sync_copy(z_hbm, r0.at[pl.ds(0, SZ)])
        for t in range(RT // SZ):
            pltpu.sync_copy(r0.at[pl.ds(0, SZ)],
                            acc.at[pl.ds(s * RT + t * SZ, SZ)])
        plsc.subcore_barrier()

        def gath(chunk, b):
            return pltpu.make_async_copy(h_hbm.at[srcs_v.at[chunk]],
                                         R[b], GS[b])

        def scat(chunk, b):
            return pltpu.make_async_copy(R[b], acc.at[dsts_v.at[chunk]],
                                         SS[b])

        def stepd(j, b):
            gath(j, b).wait()
            scat(j, b).start(add=True)
            scat(j - 1, (b + 2) % 3).wait()
            gath(j + 2, (b + 2) % 3).start()

        def body(t, carry):
            j = 1 + 3 * t
            stepd(j, 1)
            stepd(j + 1, 2)
            stepd(j + 2, 0)
            return carry

        for ph in range(PHN):
            base = wid * PHN + ph
            pltpu.sync_copy(src_hbm.at[base], srcs_v)
            pltpu.sync_copy(dst_hbm.at[base], dsts_v)
            gath(0, 0).start()
            gath(1, 1).start()
            gath(0, 0).wait()
            scat(0, 0).start(add=True)
            gath(2, 2).start()
            lax.fori_loop(0, (NPH - 5) // 3, body, 0)
            stepd(NPH - 4, (NPH - 4) % 3)
            stepd(NPH - 3, (NPH - 3) % 3)
            for j in (NPH - 2, NPH - 1):
                gath(j, j % 3).wait()
                scat(j, j % 3).start(add=True)
            for j in (NPH - 3, NPH - 2, NPH - 1):
                scat(j, j % 3).wait()
        plsc.subcore_barrier()
        for t in range(RT // SZ):
            pltpu.sync_copy(acc.at[pl.ds(s * RT + t * SZ, SZ)],
                            r0.at[pl.ds(0, SZ)])
            pltpu.sync_copy(r0.at[pl.ds(0, SZ)],
                            out_hbm.at[c, pl.ds(s * RT + t * SZ, SZ)])

    return seg


def _make_segq_kernel(N, NP, RT, EPT, NCH, G):
    mesh = plsc.VectorSubcoreMesh(core_axis_name="c", subcore_axis_name="s")
    NG = NCH // G

    @functools.partial(
        pl.kernel,
        mesh=mesh,
        out_type=jax.ShapeDtypeStruct((NC * NP,), jnp.float32),
        scratch_types=[
            pltpu.VMEM((NCH, K), jnp.int32),
            pltpu.VMEM((NCH, K), jnp.int32),
            pltpu.VMEM((2, G, K), jnp.float32),
            pltpu.VMEM((SZ,), jnp.float32),
            pltpu.VMEM_SHARED((NP,), jnp.float32),
            pltpu.SemaphoreType.DMA,
            pltpu.SemaphoreType.DMA,
            pltpu.SemaphoreType.DMA,
            pltpu.SemaphoreType.DMA,
        ],
    )
    def segq(p_hbm, src_hbm, dst_hbm, z_hbm, out_hbm,
             srcs_v, dsts_v, vals, stz, qacc, semg0, semg1, sems0, sems1):
        c = lax.axis_index("c")
        s = lax.axis_index("s")
        wid = s * NC + c
        pltpu.sync_copy(z_hbm, stz)
        for t in range(RT // SZ):
            pltpu.sync_copy(stz, qacc.at[pl.ds(s * RT + t * SZ, SZ)])
        pltpu.sync_copy(src_hbm.at[wid], srcs_v)
        pltpu.sync_copy(dst_hbm.at[wid], dsts_v)
        plsc.subcore_barrier()
        semg = (semg0, semg1)
        sems = (sems0, sems1)

        def gath(chunk, par, j):
            return pltpu.make_async_copy(
                p_hbm.at[srcs_v.at[chunk]], vals.at[par, j], semg[par])

        def scat(chunk, par, j):
            return pltpu.make_async_copy(
                vals.at[par, j], qacc.at[dsts_v.at[chunk]], sems[par])

        def fire_gath(g, par):
            for j in range(G):
                gath(g * G + j, par, j).start()

        def drain_gath(g, par):
            for j in range(G):
                gath(g * G + j, par, j).wait()

        def fire_scat(g, par):
            for j in range(G):
                scat(g * G + j, par, j).start(add=True)

        def drain_scat(g, par):
            for j in range(G):
                scat(g * G + j, par, j).wait()

        fire_gath(0, 0)

        def body(t, carry):
            g0 = 2 * t
            drain_gath(g0, 0)
            fire_scat(g0, 0)
            fire_gath(g0 + 1, 1)
            drain_gath(g0 + 1, 1)
            fire_scat(g0 + 1, 1)
            drain_scat(g0, 0)
            fire_gath(g0 + 2, 0)
            drain_scat(g0 + 1, 1)
            return carry

        if NG > 2:
            lax.fori_loop(0, NG // 2 - 1, body, 0)
        g0 = NG - 2
        drain_gath(g0, 0)
        fire_scat(g0, 0)
        fire_gath(g0 + 1, 1)
        drain_gath(g0 + 1, 1)
        fire_scat(g0 + 1, 1)
        drain_scat(g0, 0)
        drain_scat(g0 + 1, 1)
        plsc.subcore_barrier()
        for t in range(RT // SZ):
            pltpu.sync_copy(qacc.at[pl.ds(s * RT + t * SZ, SZ)], stz)
            pltpu.sync_copy(stz, out_hbm.at[pl.ds(c * NP + s * RT + t * SZ, SZ)])

    return segq


def _dot(a, b):
    return jnp.dot(a, b, preferred_element_type=jnp.float32,
                   precision=lax.Precision.HIGHEST)


def _enc_body(x_ref, w_ref, b_ref, o_ref):
    o_ref[...] = _dot(x_ref[...], w_ref[...]) + b_ref[...]


def _mid_body(h_ref, ma_ref, mb_ref, ws_ref, wn_ref, b_ref, wc_ref, o_ref):
    m = ma_ref[0] + mb_ref[0]
    h1 = _dot(h_ref[...], ws_ref[...]) + _dot(m, wn_ref[...])
    h1 = jnp.maximum(h1 + b_ref[...], 0.0)
    o_ref[...] = _dot(h1, wc_ref[...])


@jax.jit
def kernel(x, edge_index, W_enc, b_enc, W_self0, W_nei0, b0,
           W_self1, W_nei1, b1, W_head, b_head, num_seed):
    N, C = x.shape
    E = edge_index.shape[1]
    RT = _rup(-(-N // NS), SZ)
    NP = RT * NS
    EPAD = _rup(E, NW * K * NPH)
    if EPAD != E and NP == N:
        RT += SZ
        NP = RT * NS
    EPT = EPAD // NW
    NCH = EPT // K

    src = edge_index[0]
    dst = edge_index[1]
    if EPAD != E:
        src = jnp.concatenate([src, jnp.zeros((EPAD - E,), jnp.int32)])
        dst = jnp.concatenate([dst, jnp.full((EPAD - E,), N, jnp.int32)])
    srcp = src.reshape(NW * (NCH // NPH), NPH, K)
    dstp = dst.reshape(NW * (NCH // NPH), NPH, K)
    src = src.reshape(NW, NCH, K)
    dst = dst.reshape(NW, NCH, K)

    BR = 1000 if N % 1000 == 0 else 8
    NB = N // BR
    row_spec = pl.BlockSpec((BR, C), lambda i: (i, 0))
    w_spec = pl.BlockSpec((C, C), lambda i: (0, 0))
    b_spec = pl.BlockSpec((1, C), lambda i: (0, 0))

    h = pl.pallas_call(
        _enc_body,
        grid=(NB,),
        in_specs=[row_spec, w_spec, b_spec],
        out_specs=row_spec,
        out_shape=jax.ShapeDtypeStruct((N, C), jnp.float32),
    )(x, W_enc, b_enc.reshape(1, C))

    zrow = jnp.zeros((SZ, C), jnp.float32)
    seg = _make_seg_kernel(N, C, NP, RT, EPT, NCH)
    msg = seg(h, srcp, dstp, zrow)

    wnh = W_nei1 @ W_head
    wsh = W_self1 @ W_head
    Wc = jnp.concatenate([wnh, wsh], axis=1)
    ps = pl.pallas_call(
        _mid_body,
        grid=(NB,),
        in_specs=[row_spec,
                  pl.BlockSpec((1, BR, C), lambda i: (0, i, 0)),
                  pl.BlockSpec((1, BR, C), lambda i: (1, i, 0)),
                  w_spec, w_spec, b_spec,
                  pl.BlockSpec((C, 2), lambda i: (0, 0))],
        out_specs=pl.BlockSpec((BR, 2), lambda i: (i, 0)),
        out_shape=jax.ShapeDtypeStruct((N, 2), jnp.float32),
    )(h, msg, msg, W_self0, W_nei0, b0.reshape(1, C), Wc)

    p = ps[:, 0] + jnp.float32(0.0)
    s_full = ps[:, 1]
    zq = jnp.zeros((SZ,), jnp.float32)
    segq = _make_segq_kernel(N, NP, RT, EPT, NCH, 10)
    q = segq(p, src, dst, zq).reshape(NC, NP)

    tot = s_full + q[0, :N] + q[1, :N]
    seed = lax.dynamic_slice(tot, (num_seed - 1024,), (1024,))
    return seed[:, None] + (b1 @ W_head)[None, :] + b_head[None, :]


# speedup vs baseline: 1.1004x; 1.1004x over previous
import functools

import jax
import jax.numpy as jnp
from jax import lax
from jax.experimental import pallas as pl
from jax.experimental.pallas import tpu as pltpu
from jax.experimental.pallas import tpu_sc as plsc

NC = 2
NS = 16
NW = NC * NS
K = 100
SZ = 80
NPH = 20


def _rup(a, m):
    return (a + m - 1) // m * m


def _make_seg_kernel(N, C, NP, RT, EPT, NCH):
    mesh = plsc.VectorSubcoreMesh(core_axis_name="c", subcore_axis_name="s")
    PHN = NCH // NPH

    @functools.partial(
        pl.kernel,
        mesh=mesh,
        out_type=jax.ShapeDtypeStruct((NC, NP, C), jnp.float32),
        scratch_types=[
            pltpu.VMEM((NPH, K), jnp.int32),
            pltpu.VMEM((NPH, K), jnp.int32),
            pltpu.VMEM((K, C), jnp.float32),
            pltpu.VMEM((K, C), jnp.float32),
            pltpu.VMEM((K, C), jnp.float32),
            pltpu.VMEM_SHARED((NP, C), jnp.float32),
            pltpu.SemaphoreType.DMA,
            pltpu.SemaphoreType.DMA,
            pltpu.SemaphoreType.DMA,
            pltpu.SemaphoreType.DMA,
            pltpu.SemaphoreType.DMA,
            pltpu.SemaphoreType.DMA,
        ],
    )
    def seg(h_hbm, src_hbm, dst_hbm, z_hbm, out_hbm, srcs_v, dsts_v,
            r0, r1, r2, acc, g0, g1, g2, s0, s1, s2):
        c = lax.axis_index("c")
        s = lax.axis_index("s")
        wid = s * NC + c
        R = (r0, r1, r2)
        GS = (g0, g1, g2)
        SS = (s0, s1, s2)
        pltpu.sync_copy(z_hbm, r0.at[pl.ds(0, SZ)])
        for t in range(RT // SZ):
            pltpu.sync_copy(r0.at[pl.ds(0, SZ)],
                            acc.at[pl.ds(s * RT + t * SZ, SZ)])
        plsc.subcore_barrier()

        def gath(chunk, b):
            return pltpu.make_async_copy(h_hbm.at[srcs_v.at[chunk]],
                                         R[b], GS[b])

        def scat(chunk, b):
            return pltpu.make_async_copy(R[b], acc.at[dsts_v.at[chunk]],
                                         SS[b])

        def stepd(j, b):
            gath(j, b).wait()
            scat(j, b).start(add=True)
            scat(j - 1, (b + 2) % 3).wait()
            gath(j + 2, (b + 2) % 3).start()

        def body(t, carry):
            j = 1 + 3 * t
            stepd(j, 1)
            stepd(j + 1, 2)
            stepd(j + 2, 0)
            return carry

        for ph in range(PHN):
            base = wid * PHN + ph
            pltpu.sync_copy(src_hbm.at[base], srcs_v)
            pltpu.sync_copy(dst_hbm.at[base], dsts_v)
            gath(0, 0).start()
            gath(1, 1).start()
            gath(0, 0).wait()
            scat(0, 0).start(add=True)
            gath(2, 2).start()
            lax.fori_loop(0, (NPH - 5) // 3, body, 0)
            stepd(NPH - 4, (NPH - 4) % 3)
            stepd(NPH - 3, (NPH - 3) % 3)
            for j in (NPH - 2, NPH - 1):
                gath(j, j % 3).wait()
                scat(j, j % 3).start(add=True)
            for j in (NPH - 3, NPH - 2, NPH - 1):
                scat(j, j % 3).wait()
        plsc.subcore_barrier()
        for t in range(RT // SZ):
            pltpu.sync_copy(acc.at[pl.ds(s * RT + t * SZ, SZ)],
                            r0.at[pl.ds(0, SZ)])
            pltpu.sync_copy(r0.at[pl.ds(0, SZ)],
                            out_hbm.at[c, pl.ds(s * RT + t * SZ, SZ)])

    return seg


def _make_segq_kernel(N, NP, RT, EPT, NCH, G, KQ):
    mesh = plsc.VectorSubcoreMesh(core_axis_name="c", subcore_axis_name="s")
    NG = NCH // G

    @functools.partial(
        pl.kernel,
        mesh=mesh,
        out_type=jax.ShapeDtypeStruct((NC * NP,), jnp.float32),
        scratch_types=[
            pltpu.VMEM((NCH, KQ), jnp.int32),
            pltpu.VMEM((NCH, KQ), jnp.int32),
            pltpu.VMEM((2, G, KQ), jnp.float32),
            pltpu.VMEM((SZ,), jnp.float32),
            pltpu.VMEM_SHARED((NP,), jnp.float32),
            pltpu.SemaphoreType.DMA,
            pltpu.SemaphoreType.DMA,
            pltpu.SemaphoreType.DMA,
            pltpu.SemaphoreType.DMA,
        ],
    )
    def segq(p_hbm, src_hbm, dst_hbm, z_hbm, out_hbm,
             srcs_v, dsts_v, vals, stz, qacc, semg0, semg1, sems0, sems1):
        c = lax.axis_index("c")
        s = lax.axis_index("s")
        wid = s * NC + c
        pltpu.sync_copy(z_hbm, stz)
        for t in range(RT // SZ):
            pltpu.sync_copy(stz, qacc.at[pl.ds(s * RT + t * SZ, SZ)])
        pltpu.sync_copy(src_hbm.at[wid], srcs_v)
        pltpu.sync_copy(dst_hbm.at[wid], dsts_v)
        plsc.subcore_barrier()
        semg = (semg0, semg1)
        sems = (sems0, sems1)

        def gath(chunk, par, j):
            return pltpu.make_async_copy(
                p_hbm.at[srcs_v.at[chunk]], vals.at[par, j], semg[par])

        def scat(chunk, par, j):
            return pltpu.make_async_copy(
                vals.at[par, j], qacc.at[dsts_v.at[chunk]], sems[par])

        def fire_gath(g, par):
            for j in range(G):
                gath(g * G + j, par, j).start()

        def drain_gath(g, par):
            for j in range(G):
                gath(g * G + j, par, j).wait()

        def fire_scat(g, par):
            for j in range(G):
                scat(g * G + j, par, j).start(add=True)

        def drain_scat(g, par):
            for j in range(G):
                scat(g * G + j, par, j).wait()

        fire_gath(0, 0)

        def body(t, carry):
            g0 = 2 * t
            drain_gath(g0, 0)
            fire_scat(g0, 0)
            fire_gath(g0 + 1, 1)
            drain_gath(g0 + 1, 1)
            fire_scat(g0 + 1, 1)
            drain_scat(g0, 0)
            fire_gath(g0 + 2, 0)
            drain_scat(g0 + 1, 1)
            return carry

        if NG > 2:
            lax.fori_loop(0, NG // 2 - 1, body, 0)
        g0 = NG - 2
        drain_gath(g0, 0)
        fire_scat(g0, 0)
        fire_gath(g0 + 1, 1)
        drain_gath(g0 + 1, 1)
        fire_scat(g0 + 1, 1)
        drain_scat(g0, 0)
        drain_scat(g0 + 1, 1)
        plsc.subcore_barrier()
        for t in range(RT // SZ):
            pltpu.sync_copy(qacc.at[pl.ds(s * RT + t * SZ, SZ)], stz)
            pltpu.sync_copy(stz, out_hbm.at[pl.ds(c * NP + s * RT + t * SZ, SZ)])

    return segq


def _dot(a, b):
    return jnp.dot(a, b, preferred_element_type=jnp.float32,
                   precision=lax.Precision.HIGHEST)


def _enc_body(x_ref, w_ref, b_ref, o_ref):
    o_ref[...] = _dot(x_ref[...], w_ref[...]) + b_ref[...]


def _mid_body(h_ref, ma_ref, mb_ref, ws_ref, wn_ref, b_ref, wc_ref, o_ref):
    m = ma_ref[0] + mb_ref[0]
    h1 = _dot(h_ref[...], ws_ref[...]) + _dot(m, wn_ref[...])
    h1 = jnp.maximum(h1 + b_ref[...], 0.0)
    o_ref[...] = _dot(h1, wc_ref[...])


@jax.jit
def kernel(x, edge_index, W_enc, b_enc, W_self0, W_nei0, b0,
           W_self1, W_nei1, b1, W_head, b_head, num_seed):
    N, C = x.shape
    E = edge_index.shape[1]
    RT = _rup(-(-N // NS), SZ)
    NP = RT * NS
    EPAD = _rup(E, NW * K * NPH)
    if EPAD != E and NP == N:
        RT += SZ
        NP = RT * NS
    EPT = EPAD // NW
    NCH = EPT // K

    src = edge_index[0]
    dst = edge_index[1]
    if EPAD != E:
        src = jnp.concatenate([src, jnp.zeros((EPAD - E,), jnp.int32)])
        dst = jnp.concatenate([dst, jnp.full((EPAD - E,), N, jnp.int32)])
    srcp = src.reshape(NW * (NCH // NPH), NPH, K)
    dstp = dst.reshape(NW * (NCH // NPH), NPH, K)
    src = src.reshape(NW, NCH, K)
    dst = dst.reshape(NW, NCH, K)

    BR = 1000 if N % 1000 == 0 else 8
    NB = N // BR
    row_spec = pl.BlockSpec((BR, C), lambda i: (i, 0))
    w_spec = pl.BlockSpec((C, C), lambda i: (0, 0))
    b_spec = pl.BlockSpec((1, C), lambda i: (0, 0))

    h = pl.pallas_call(
        _enc_body,
        grid=(NB,),
        in_specs=[row_spec, w_spec, b_spec],
        out_specs=row_spec,
        out_shape=jax.ShapeDtypeStruct((N, C), jnp.float32),
    )(x, W_enc, b_enc.reshape(1, C))

    zrow = jnp.zeros((SZ, C), jnp.float32)
    seg = _make_seg_kernel(N, C, NP, RT, EPT, NCH)
    msg = seg(h, srcp, dstp, zrow)

    wnh = W_nei1 @ W_head
    wsh = W_self1 @ W_head
    Wc = jnp.concatenate([wnh, wsh], axis=1)
    ps = pl.pallas_call(
        _mid_body,
        grid=(NB,),
        in_specs=[row_spec,
                  pl.BlockSpec((1, BR, C), lambda i: (0, i, 0)),
                  pl.BlockSpec((1, BR, C), lambda i: (1, i, 0)),
                  w_spec, w_spec, b_spec,
                  pl.BlockSpec((C, 2), lambda i: (0, 0))],
        out_specs=pl.BlockSpec((BR, 2), lambda i: (i, 0)),
        out_shape=jax.ShapeDtypeStruct((N, 2), jnp.float32),
    )(h, msg, msg, W_self0, W_nei0, b0.reshape(1, C), Wc)

    p = ps[:, 0] + jnp.float32(0.0)
    s_full = ps[:, 1]
    zq = jnp.zeros((SZ,), jnp.float32)
    KQ = 125
    srcq = src.reshape(NW, EPT // KQ, KQ)
    dstq = dst.reshape(NW, EPT // KQ, KQ)
    segq = _make_segq_kernel(N, NP, RT, EPT, EPT // KQ, 8, KQ)
    q = segq(p, srcq, dstq, zq).reshape(NC, NP)

    tot = s_full + q[0, :N] + q[1, :N]
    seed = lax.dynamic_slice(tot, (num_seed - 1024,), (1024,))
    return seed[:, None] + (b1 @ W_head)[None, :] + b_head[None, :]
